# baseline (speedup 1.0000x reference)
import functools

import jax
import jax.numpy as jnp
from jax import lax
from jax.experimental import pallas as pl
from jax.experimental.pallas import tpu as pltpu
from jax.experimental.pallas import tpu_sc as plsc

N_NODES = 10000
F_IN = 256
HEADS = 8
OUT1 = 8
C1 = HEADS * OUT1
NUM_CLASSES = 40

NP = 10112
NSUB = 16
NW = 32
ROWS_PER_SUB = NP // NSUB
EDGE_CHUNK = 64
B_W = 5376
EP = NW * B_W

R_TC = 632
_HI = lax.Precision.HIGHEST


def _make_edge_kernel(src_off, dst_off, nmsg):
    ee_off = 16 * nmsg
    mesh = plsc.VectorSubcoreMesh(core_axis_name="c", subcore_axis_name="s")

    @functools.partial(
        pl.kernel,
        mesh=mesh,
        out_type=jax.ShapeDtypeStruct((2 * NP, 128), jnp.float32),
        scratch_types=[
            pltpu.VMEM((EDGE_CHUNK,), jnp.int32),
            pltpu.VMEM((EDGE_CHUNK,), jnp.int32),
            pltpu.VMEM((EDGE_CHUNK,), jnp.int32),
            pltpu.VMEM((EDGE_CHUNK,), jnp.int32),
            pltpu.VMEM((EDGE_CHUNK, 128), jnp.float32),
            pltpu.VMEM((EDGE_CHUNK, 128), jnp.float32),
            pltpu.VMEM((EDGE_CHUNK, 128), jnp.float32),
            pltpu.VMEM((EDGE_CHUNK, 128), jnp.float32),
            pltpu.VMEM((EDGE_CHUNK, 128), jnp.float32),
            pltpu.VMEM((EDGE_CHUNK, 128), jnp.float32),
            pltpu.VMEM_SHARED((NP, 128), jnp.float32),
            pltpu.SemaphoreType.DMA,
            pltpu.SemaphoreType.DMA,
            pltpu.SemaphoreType.DMA,
            pltpu.SemaphoreType.DMA,
            pltpu.SemaphoreType.DMA,
            pltpu.SemaphoreType.DMA,
        ],
    )
    def edge_kernel(src_hbm, dst_hbm, rec_hbm, out_hbm,
                    idx_s0, idx_d0, idx_s1, idx_d1,
                    rec_v0, recd_v0, rec_v1, recd_v1, out_v0, out_v1, acc,
                    sg0a, sg0b, sg1a, sg1b, ss0, ss1):
        c = lax.axis_index("c")
        s = lax.axis_index("s")
        row0 = s * ROWS_PER_SUB
        zero16 = jnp.zeros((16,), jnp.float32)

        @pl.loop(0, EDGE_CHUNK)
        def _zrow(r):
            for j in range(8):
                out_v0[r, pl.ds(16 * j, 16)] = zero16
                out_v1[r, pl.ds(16 * j, 16)] = zero16

        @pl.loop(0, ROWS_PER_SUB, step=8)
        def _init(j):
            pltpu.sync_copy(out_v0.at[pl.ds(0, 8)],
                            acc.at[pl.ds(row0 + j, 8)])

        plsc.subcore_barrier()
        base_w = (c * NSUB + s) * B_W

        def _compute(rec, recd, out):
            @pl.loop(0, EDGE_CHUNK)
            def _edge(e):
                av = rec[e, pl.ds(src_off, 16)]
                ad = recd[e, pl.ds(dst_off, 16)]
                t0 = av + ad
                al = jnp.minimum(jnp.maximum(t0, 0.2 * t0), 75.0)
                ee = jnp.exp(al)
                out[e, pl.ds(ee_off, 16)] = ee
                for j in range(nmsg):
                    out[e, pl.ds(16 * j, 16)] = rec[e, pl.ds(16 * j, 16)] * ee

        @pl.loop(0, B_W, step=2 * EDGE_CHUNK)
        def _pair(t):
            b0 = base_w + t
            b1 = b0 + EDGE_CHUNK
            pltpu.sync_copy(src_hbm.at[pl.ds(b0, EDGE_CHUNK)], idx_s0)
            pltpu.sync_copy(dst_hbm.at[pl.ds(b0, EDGE_CHUNK)], idx_d0)
            pltpu.sync_copy(src_hbm.at[pl.ds(b1, EDGE_CHUNK)], idx_s1)
            pltpu.sync_copy(dst_hbm.at[pl.ds(b1, EDGE_CHUNK)], idx_d1)
            g0a = pltpu.async_copy(rec_hbm.at[idx_s0], rec_v0, sg0a)
            g0b = pltpu.async_copy(rec_hbm.at[idx_d0], recd_v0, sg0b)
            g1a = pltpu.async_copy(rec_hbm.at[idx_s1], rec_v1, sg1a)
            g1b = pltpu.async_copy(rec_hbm.at[idx_d1], recd_v1, sg1b)
            g0a.wait()
            g0b.wait()
            _compute(rec_v0, recd_v0, out_v0)
            s0 = pltpu.async_copy(out_v0, acc.at[idx_d0], ss0, add=True)
            g1a.wait()
            g1b.wait()
            _compute(rec_v1, recd_v1, out_v1)
            s1 = pltpu.async_copy(out_v1, acc.at[idx_d1], ss1, add=True)
            s0.wait()
            s1.wait()

        plsc.subcore_barrier()
        pltpu.sync_copy(acc.at[pl.ds(row0, ROWS_PER_SUB)],
                        out_hbm.at[pl.ds(c * NP + row0, ROWS_PER_SUB)])

    return edge_kernel


_edge_kernel_l1 = _make_edge_kernel(64, 80, 4)
_edge_kernel_l2 = _make_edge_kernel(40, 56, 3)


def _prep1_body(x_ref, wrec_ref, rec_ref):
    rec_ref[...] = jnp.dot(x_ref[...], wrec_ref[...], precision=_HI,
                           preferred_element_type=jnp.float32)


def _merge1_body(p_ref, s1_ref, b1_ref, w2_ref, rec2_ref):
    acc = p_ref[0] + p_ref[1]
    num = acc[:, :C1]
    den8 = acc[:, C1:C1 + HEADS]
    denb = jnp.dot(den8, s1_ref[...], precision=_HI,
                   preferred_element_type=jnp.float32)
    h1 = num / (denb + 1e-16) + b1_ref[...]
    h1 = jnp.where(h1 > 0, h1, jnp.exp(jnp.minimum(h1, 0.0)) - 1.0)
    rec2_ref[...] = jnp.dot(h1, w2_ref[...], precision=_HI,
                            preferred_element_type=jnp.float32)


def _merge2_body(p_ref, s2_ref, b2_ref, out_ref):
    acc = p_ref[0] + p_ref[1]
    num = acc[:, :NUM_CLASSES]
    den8 = acc[:, 48:56]
    denb = jnp.dot(den8, s2_ref[...], precision=_HI,
                   preferred_element_type=jnp.float32)
    logits = num / (denb + 1e-16) + b2_ref[...]
    m = jnp.max(logits, axis=1, keepdims=True)
    z = logits - m
    lse = jnp.log(jnp.sum(jnp.exp(z), axis=1, keepdims=True))
    out_ref[...] = z - lse


def kernel(x, edge_index, W1, att_src1, att_dst1, b1, W2, att_src2, att_dst2, b2):
    f32 = jnp.float32
    n = N_NODES

    W1p = W1.reshape(F_IN, HEADS, OUT1).transpose(0, 2, 1).reshape(F_IN, C1)
    wa = (W1.reshape(F_IN, HEADS, OUT1) * att_src1[None]).sum(-1)
    wd = (W1.reshape(F_IN, HEADS, OUT1) * att_dst1[None]).sum(-1)
    wrec1 = jnp.concatenate([W1p, wa, wa, wd, wd,
                             jnp.zeros((F_IN, 32), f32)],
                            axis=1).astype(f32)
    b1p = b1.reshape(HEADS, OUT1).transpose(1, 0).reshape(1, C1).astype(f32)

    W2p = W2.reshape(HEADS, OUT1, NUM_CLASSES).transpose(1, 0, 2).reshape(
        C1, NUM_CLASSES)
    va2 = W2p @ att_src2.reshape(-1)
    vd2 = W2p @ att_dst2.reshape(-1)
    w2cat = jnp.concatenate(
        [W2p, jnp.tile(va2[:, None], (1, 16)), jnp.tile(vd2[:, None], (1, 16)),
         jnp.zeros((C1, 56), f32)],
        axis=1).astype(f32)
    b2row = b2.reshape(1, NUM_CLASSES).astype(f32)

    heads_of = jnp.arange(C1, dtype=jnp.int32) % HEADS
    s1 = (heads_of[None, :] == jnp.arange(HEADS, dtype=jnp.int32)[:, None]
          ).astype(f32)
    s2 = jnp.full((8, NUM_CLASSES), 1.0 / 8.0, f32)

    x_pad = jnp.zeros((NP, F_IN), f32).at[:n].set(x.astype(f32))

    ar = jnp.arange(n, dtype=jnp.int32)
    npad = EP - n - edge_index.shape[1]
    pad_idx = n + jnp.arange(npad, dtype=jnp.int32) % (NP - n)
    src = jnp.concatenate([edge_index[0].astype(jnp.int32), ar, pad_idx])
    dst = jnp.concatenate([edge_index[1].astype(jnp.int32), ar, pad_idx])

    grid = (NP // R_TC,)

    rec1 = pl.pallas_call(
        _prep1_body,
        grid=grid,
        in_specs=[
            pl.BlockSpec((R_TC, F_IN), lambda i: (i, 0)),
            pl.BlockSpec((F_IN, 128), lambda i: (0, 0)),
        ],
        out_specs=pl.BlockSpec((R_TC, 128), lambda i: (i, 0)),
        out_shape=jax.ShapeDtypeStruct((NP, 128), f32),
    )(x_pad, wrec1)

    part1 = _edge_kernel_l1(src, dst, rec1).reshape(2, NP, 128)

    rec2 = pl.pallas_call(
        _merge1_body,
        grid=grid,
        in_specs=[
            pl.BlockSpec((2, R_TC, 128), lambda i: (0, i, 0)),
            pl.BlockSpec((HEADS, C1), lambda i: (0, 0)),
            pl.BlockSpec((1, C1), lambda i: (0, 0)),
            pl.BlockSpec((C1, 128), lambda i: (0, 0)),
        ],
        out_specs=pl.BlockSpec((R_TC, 128), lambda i: (i, 0)),
        out_shape=jax.ShapeDtypeStruct((NP, 128), f32),
    )(part1, s1, b1p, w2cat)

    part2 = _edge_kernel_l2(src, dst, rec2).reshape(2, NP, 128)

    out_full = pl.pallas_call(
        _merge2_body,
        grid=grid,
        in_specs=[
            pl.BlockSpec((2, R_TC, 128), lambda i: (0, i, 0)),
            pl.BlockSpec((8, NUM_CLASSES), lambda i: (0, 0)),
            pl.BlockSpec((1, NUM_CLASSES), lambda i: (0, 0)),
        ],
        out_specs=pl.BlockSpec((R_TC, NUM_CLASSES), lambda i: (i, 0)),
        out_shape=jax.ShapeDtypeStruct((NP, NUM_CLASSES), f32),
    )(part2, s2, b2row)

    return out_full[:n]


# speedup vs baseline: 1.1214x; 1.1214x over previous
import functools

import jax
import jax.numpy as jnp
from jax import lax
from jax.experimental import pallas as pl
from jax.experimental.pallas import tpu as pltpu
from jax.experimental.pallas import tpu_sc as plsc

N_NODES = 10000
F_IN = 256
HEADS = 8
OUT1 = 8
C1 = HEADS * OUT1
NUM_CLASSES = 40

NP = 10112
NSUB = 16
NW = 32
ROWS_PER_SUB = NP // NSUB
EDGE_CHUNK = 128
B_W = 5376
EP = NW * B_W

R_TC = 632
_HI = lax.Precision.HIGHEST


def _make_edge_kernel(src_off, dst_off, nmsg):
    ee_off = 16 * nmsg
    mesh = plsc.VectorSubcoreMesh(core_axis_name="c", subcore_axis_name="s")

    @functools.partial(
        pl.kernel,
        mesh=mesh,
        out_type=jax.ShapeDtypeStruct((2 * NP, 128), jnp.float32),
        scratch_types=[
            pltpu.VMEM((3, EDGE_CHUNK), jnp.int32),
            pltpu.VMEM((3, EDGE_CHUNK), jnp.int32),
            pltpu.VMEM((EDGE_CHUNK, 128), jnp.float32),
            pltpu.VMEM((EDGE_CHUNK, 128), jnp.float32),
            pltpu.VMEM((EDGE_CHUNK, 128), jnp.float32),
            pltpu.VMEM_SHARED((NP, 128), jnp.float32),
            pltpu.SemaphoreType.DMA,
            pltpu.SemaphoreType.DMA,
            pltpu.SemaphoreType.DMA,
            pltpu.SemaphoreType.DMA,
            pltpu.SemaphoreType.DMA,
        ],
    )
    def edge_kernel(src_hbm, dst_hbm, rec_hbm, out_hbm,
                    idx_s, idx_d, rec_v0, rec_v1, recd_v, acc,
                    sga, sgb, sia, sib, ss):
        c = lax.axis_index("c")
        s = lax.axis_index("s")
        row0 = s * ROWS_PER_SUB
        zero16 = jnp.zeros((16,), jnp.float32)
        rec_bufs = (rec_v0, rec_v1)

        @pl.loop(0, 8)
        def _zrow(r):
            for j in range(8):
                rec_v0[r, pl.ds(16 * j, 16)] = zero16

        @pl.loop(0, ROWS_PER_SUB, step=8)
        def _init(j):
            pltpu.sync_copy(rec_v0.at[pl.ds(0, 8)],
                            acc.at[pl.ds(row0 + j, 8)])

        plsc.subcore_barrier()
        base_w = (c * NSUB + s) * B_W

        def _compute(rec, recd):
            @pl.loop(0, EDGE_CHUNK)
            def _edge(e):
                av = rec[e, pl.ds(src_off, 16)]
                ad = recd[e, pl.ds(dst_off, 16)]
                t0 = av + ad
                al = jnp.minimum(jnp.maximum(t0, 0.2 * t0), 75.0)
                ee = jnp.exp(al)
                for j in range(nmsg):
                    rec[e, pl.ds(16 * j, 16)] = rec[e, pl.ds(16 * j, 16)] * ee
                rec[e, pl.ds(ee_off, 16)] = ee

        n_chunks = B_W // EDGE_CHUNK

        def _load_idx(ci):
            k = ci % 3
            return (
                pltpu.async_copy(
                    src_hbm.at[pl.ds(base_w + ci * EDGE_CHUNK, EDGE_CHUNK)],
                    idx_s.at[k], sia),
                pltpu.async_copy(
                    dst_hbm.at[pl.ds(base_w + ci * EDGE_CHUNK, EDGE_CHUNK)],
                    idx_d.at[k], sib),
            )

        def _fire_gathers(ci):
            k = ci % 3
            return (
                pltpu.async_copy(rec_hbm.at[idx_s.at[k]], rec_bufs[ci % 2],
                                 sga),
                pltpu.async_copy(rec_hbm.at[idx_d.at[k]], recd_v, sgb),
            )

        idx_h = {0: _load_idx(0)}
        idx_h[0][0].wait()
        idx_h[0][1].wait()
        g_h = {0: _fire_gathers(0)}
        idx_h[1] = _load_idx(1)
        sc_h = {}
        for ci in range(n_chunks):
            g_h[ci][0].wait()
            g_h[ci][1].wait()
            _compute(rec_bufs[ci % 2], recd_v)
            if ci > 0:
                sc_h[ci - 1].wait()
            if ci + 1 < n_chunks:
                idx_h[ci + 1][0].wait()
                idx_h[ci + 1][1].wait()
                g_h[ci + 1] = _fire_gathers(ci + 1)
            sc_h[ci] = pltpu.async_copy(
                rec_bufs[ci % 2], acc.at[idx_d.at[ci % 3]], ss, add=True)
            if ci + 2 < n_chunks:
                idx_h[ci + 2] = _load_idx(ci + 2)
        sc_h[n_chunks - 1].wait()

        plsc.subcore_barrier()
        pltpu.sync_copy(acc.at[pl.ds(row0, ROWS_PER_SUB)],
                        out_hbm.at[pl.ds(c * NP + row0, ROWS_PER_SUB)])

    return edge_kernel


_edge_kernel_l1 = _make_edge_kernel(64, 80, 4)
_edge_kernel_l2 = _make_edge_kernel(40, 56, 3)


def _prep1_body(x_ref, wrec_ref, rec_ref):
    rec_ref[...] = jnp.dot(x_ref[...], wrec_ref[...], precision=_HI,
                           preferred_element_type=jnp.float32)


def _merge1_body(p_ref, s1_ref, b1_ref, w2_ref, rec2_ref):
    acc = p_ref[0] + p_ref[1]
    num = acc[:, :C1]
    den8 = acc[:, C1:C1 + HEADS]
    denb = jnp.dot(den8, s1_ref[...], precision=_HI,
                   preferred_element_type=jnp.float32)
    h1 = num / (denb + 1e-16) + b1_ref[...]
    h1 = jnp.where(h1 > 0, h1, jnp.exp(jnp.minimum(h1, 0.0)) - 1.0)
    rec2_ref[...] = jnp.dot(h1, w2_ref[...], precision=_HI,
                            preferred_element_type=jnp.float32)


def _merge2_body(p_ref, s2_ref, b2_ref, out_ref):
    acc = p_ref[0] + p_ref[1]
    num = acc[:, :NUM_CLASSES]
    den8 = acc[:, 48:56]
    denb = jnp.dot(den8, s2_ref[...], precision=_HI,
                   preferred_element_type=jnp.float32)
    logits = num / (denb + 1e-16) + b2_ref[...]
    m = jnp.max(logits, axis=1, keepdims=True)
    z = logits - m
    lse = jnp.log(jnp.sum(jnp.exp(z), axis=1, keepdims=True))
    out_ref[...] = z - lse


def kernel(x, edge_index, W1, att_src1, att_dst1, b1, W2, att_src2, att_dst2, b2):
    f32 = jnp.float32
    n = N_NODES

    W1p = W1.reshape(F_IN, HEADS, OUT1).transpose(0, 2, 1).reshape(F_IN, C1)
    wa = (W1.reshape(F_IN, HEADS, OUT1) * att_src1[None]).sum(-1)
    wd = (W1.reshape(F_IN, HEADS, OUT1) * att_dst1[None]).sum(-1)
    wrec1 = jnp.concatenate([W1p, wa, wa, wd, wd,
                             jnp.zeros((F_IN, 32), f32)],
                            axis=1).astype(f32)
    b1p = b1.reshape(HEADS, OUT1).transpose(1, 0).reshape(1, C1).astype(f32)

    W2p = W2.reshape(HEADS, OUT1, NUM_CLASSES).transpose(1, 0, 2).reshape(
        C1, NUM_CLASSES)
    va2 = W2p @ att_src2.reshape(-1)
    vd2 = W2p @ att_dst2.reshape(-1)
    w2cat = jnp.concatenate(
        [W2p, jnp.tile(va2[:, None], (1, 16)), jnp.tile(vd2[:, None], (1, 16)),
         jnp.zeros((C1, 56), f32)],
        axis=1).astype(f32)
    b2row = b2.reshape(1, NUM_CLASSES).astype(f32)

    heads_of = jnp.arange(C1, dtype=jnp.int32) % HEADS
    s1 = (heads_of[None, :] == jnp.arange(HEADS, dtype=jnp.int32)[:, None]
          ).astype(f32)
    s2 = jnp.full((8, NUM_CLASSES), 1.0 / 8.0, f32)

    x_pad = jnp.zeros((NP, F_IN), f32).at[:n].set(x.astype(f32))

    ar = jnp.arange(n, dtype=jnp.int32)
    npad = EP - n - edge_index.shape[1]
    pad_idx = n + jnp.arange(npad, dtype=jnp.int32) % (NP - n)
    src = jnp.concatenate([edge_index[0].astype(jnp.int32), ar, pad_idx])
    dst = jnp.concatenate([edge_index[1].astype(jnp.int32), ar, pad_idx])

    grid = (NP // R_TC,)

    rec1 = pl.pallas_call(
        _prep1_body,
        grid=grid,
        in_specs=[
            pl.BlockSpec((R_TC, F_IN), lambda i: (i, 0)),
            pl.BlockSpec((F_IN, 128), lambda i: (0, 0)),
        ],
        out_specs=pl.BlockSpec((R_TC, 128), lambda i: (i, 0)),
        out_shape=jax.ShapeDtypeStruct((NP, 128), f32),
    )(x_pad, wrec1)

    part1 = _edge_kernel_l1(src, dst, rec1).reshape(2, NP, 128)

    rec2 = pl.pallas_call(
        _merge1_body,
        grid=grid,
        in_specs=[
            pl.BlockSpec((2, R_TC, 128), lambda i: (0, i, 0)),
            pl.BlockSpec((HEADS, C1), lambda i: (0, 0)),
            pl.BlockSpec((1, C1), lambda i: (0, 0)),
            pl.BlockSpec((C1, 128), lambda i: (0, 0)),
        ],
        out_specs=pl.BlockSpec((R_TC, 128), lambda i: (i, 0)),
        out_shape=jax.ShapeDtypeStruct((NP, 128), f32),
    )(part1, s1, b1p, w2cat)

    part2 = _edge_kernel_l2(src, dst, rec2).reshape(2, NP, 128)

    out_full = pl.pallas_call(
        _merge2_body,
        grid=grid,
        in_specs=[
            pl.BlockSpec((2, R_TC, 128), lambda i: (0, i, 0)),
            pl.BlockSpec((8, NUM_CLASSES), lambda i: (0, 0)),
            pl.BlockSpec((1, NUM_CLASSES), lambda i: (0, 0)),
        ],
        out_specs=pl.BlockSpec((R_TC, NUM_CLASSES), lambda i: (i, 0)),
        out_shape=jax.ShapeDtypeStruct((NP, NUM_CLASSES), f32),
    )(part2, s2, b2row)

    return out_full[:n]


# speedup vs baseline: 1.4333x; 1.2782x over previous
import functools

import jax
import jax.numpy as jnp
from jax import lax
from jax.experimental import pallas as pl
from jax.experimental.pallas import tpu as pltpu
from jax.experimental.pallas import tpu_sc as plsc

N_NODES = 10000
F_IN = 256
HEADS = 8
OUT1 = 8
C1 = HEADS * OUT1
NUM_CLASSES = 40

NP = 10112
NSUB = 16
NW = 32
ROWS_PER_SUB = NP // NSUB
EDGE_CHUNK = 96
B_W = 5376
EP = NW * B_W

R_TC = 632
_HI = lax.Precision.HIGHEST


def _make_edge_kernel(src_off, dst_off, nmsg):
    ee_off = 16 * nmsg
    mesh = plsc.VectorSubcoreMesh(core_axis_name="c", subcore_axis_name="s")

    @functools.partial(
        pl.kernel,
        mesh=mesh,
        out_type=jax.ShapeDtypeStruct((2 * NP, 128), jnp.float32),
        scratch_types=[
            pltpu.VMEM((3, EDGE_CHUNK), jnp.int32),
            pltpu.VMEM((3, EDGE_CHUNK), jnp.int32),
            pltpu.VMEM((EDGE_CHUNK, 128), jnp.float32),
            pltpu.VMEM((EDGE_CHUNK, 128), jnp.float32),
            pltpu.VMEM((EDGE_CHUNK, 128), jnp.float32),
            pltpu.VMEM((EDGE_CHUNK, 128), jnp.float32),
            pltpu.VMEM_SHARED((NP, 128), jnp.float32),
            pltpu.SemaphoreType.DMA,
            pltpu.SemaphoreType.DMA,
            pltpu.SemaphoreType.DMA,
            pltpu.SemaphoreType.DMA,
            pltpu.SemaphoreType.DMA,
        ],
    )
    def edge_kernel(src_hbm, dst_hbm, rec_hbm, out_hbm,
                    idx_s, idx_d, rec_v0, rec_v1, recd_v0, recd_v1, acc,
                    sga, sgb, sia, sib, ss):
        c = lax.axis_index("c")
        s = lax.axis_index("s")
        row0 = s * ROWS_PER_SUB
        zero16 = jnp.zeros((16,), jnp.float32)
        rec_bufs = (rec_v0, rec_v1)
        recd_bufs = (recd_v0, recd_v1)

        @pl.loop(0, 8)
        def _zrow(r):
            for j in range(8):
                rec_v0[r, pl.ds(16 * j, 16)] = zero16

        @pl.loop(0, ROWS_PER_SUB, step=8)
        def _init(j):
            pltpu.sync_copy(rec_v0.at[pl.ds(0, 8)],
                            acc.at[pl.ds(row0 + j, 8)])

        plsc.subcore_barrier()
        base_w = (c * NSUB + s) * B_W

        def _compute(rec, recd):
            @pl.loop(0, EDGE_CHUNK)
            def _edge(e):
                av = rec[e, pl.ds(src_off, 16)]
                ad = recd[e, pl.ds(dst_off, 16)]
                t0 = av + ad
                al = jnp.minimum(jnp.maximum(t0, 0.2 * t0), 75.0)
                ee = jnp.exp(al)
                for j in range(nmsg):
                    rec[e, pl.ds(16 * j, 16)] = rec[e, pl.ds(16 * j, 16)] * ee
                rec[e, pl.ds(ee_off, 16)] = ee

        n_chunks = B_W // EDGE_CHUNK

        def _load_idx(ci):
            k = ci % 3
            return (
                pltpu.async_copy(
                    src_hbm.at[pl.ds(base_w + ci * EDGE_CHUNK, EDGE_CHUNK)],
                    idx_s.at[k], sia),
                pltpu.async_copy(
                    dst_hbm.at[pl.ds(base_w + ci * EDGE_CHUNK, EDGE_CHUNK)],
                    idx_d.at[k], sib),
            )

        def _fire_gathers(ci):
            k = ci % 3
            return (
                pltpu.async_copy(rec_hbm.at[idx_s.at[k]], rec_bufs[ci % 2],
                                 sga),
                pltpu.async_copy(rec_hbm.at[idx_d.at[k]], recd_bufs[ci % 2],
                                 sgb),
            )

        idx_h = {0: _load_idx(0)}
        idx_h[0][0].wait()
        idx_h[0][1].wait()
        g_h = {0: _fire_gathers(0)}
        idx_h[1] = _load_idx(1)
        sc_h = {}
        for ci in range(n_chunks):
            g_h[ci][0].wait()
            g_h[ci][1].wait()
            if ci > 0:
                sc_h[ci - 1].wait()
            if ci + 1 < n_chunks:
                idx_h[ci + 1][0].wait()
                idx_h[ci + 1][1].wait()
                g_h[ci + 1] = _fire_gathers(ci + 1)
            _compute(rec_bufs[ci % 2], recd_bufs[ci % 2])
            sc_h[ci] = pltpu.async_copy(
                rec_bufs[ci % 2], acc.at[idx_d.at[ci % 3]], ss, add=True)
            if ci + 2 < n_chunks:
                idx_h[ci + 2] = _load_idx(ci + 2)
        sc_h[n_chunks - 1].wait()

        plsc.subcore_barrier()
        pltpu.sync_copy(acc.at[pl.ds(row0, ROWS_PER_SUB)],
                        out_hbm.at[pl.ds(c * NP + row0, ROWS_PER_SUB)])

    return edge_kernel


_edge_kernel_l1 = _make_edge_kernel(64, 80, 4)
_edge_kernel_l2 = _make_edge_kernel(40, 56, 3)


def _prep1_body(x_ref, wrec_ref, rec_ref):
    rec_ref[...] = jnp.dot(x_ref[...], wrec_ref[...], precision=_HI,
                           preferred_element_type=jnp.float32)


def _merge1_body(p_ref, s1_ref, b1_ref, w2_ref, rec2_ref):
    acc = p_ref[0] + p_ref[1]
    num = acc[:, :C1]
    den8 = acc[:, C1:C1 + HEADS]
    denb = jnp.dot(den8, s1_ref[...], precision=_HI,
                   preferred_element_type=jnp.float32)
    h1 = num / (denb + 1e-16) + b1_ref[...]
    h1 = jnp.where(h1 > 0, h1, jnp.exp(jnp.minimum(h1, 0.0)) - 1.0)
    rec2_ref[...] = jnp.dot(h1, w2_ref[...], precision=_HI,
                            preferred_element_type=jnp.float32)


def _merge2_body(p_ref, s2_ref, b2_ref, out_ref):
    acc = p_ref[0] + p_ref[1]
    num = acc[:, :NUM_CLASSES]
    den8 = acc[:, 48:56]
    denb = jnp.dot(den8, s2_ref[...], precision=_HI,
                   preferred_element_type=jnp.float32)
    logits = num / (denb + 1e-16) + b2_ref[...]
    m = jnp.max(logits, axis=1, keepdims=True)
    z = logits - m
    lse = jnp.log(jnp.sum(jnp.exp(z), axis=1, keepdims=True))
    out_ref[...] = z - lse


def kernel(x, edge_index, W1, att_src1, att_dst1, b1, W2, att_src2, att_dst2, b2):
    f32 = jnp.float32
    n = N_NODES

    W1p = W1.reshape(F_IN, HEADS, OUT1).transpose(0, 2, 1).reshape(F_IN, C1)
    wa = (W1.reshape(F_IN, HEADS, OUT1) * att_src1[None]).sum(-1)
    wd = (W1.reshape(F_IN, HEADS, OUT1) * att_dst1[None]).sum(-1)
    wrec1 = jnp.concatenate([W1p, wa, wa, wd, wd,
                             jnp.zeros((F_IN, 32), f32)],
                            axis=1).astype(f32)
    b1p = b1.reshape(HEADS, OUT1).transpose(1, 0).reshape(1, C1).astype(f32)

    W2p = W2.reshape(HEADS, OUT1, NUM_CLASSES).transpose(1, 0, 2).reshape(
        C1, NUM_CLASSES)
    va2 = W2p @ att_src2.reshape(-1)
    vd2 = W2p @ att_dst2.reshape(-1)
    w2cat = jnp.concatenate(
        [W2p, jnp.tile(va2[:, None], (1, 16)), jnp.tile(vd2[:, None], (1, 16)),
         jnp.zeros((C1, 56), f32)],
        axis=1).astype(f32)
    b2row = b2.reshape(1, NUM_CLASSES).astype(f32)

    heads_of = jnp.arange(C1, dtype=jnp.int32) % HEADS
    s1 = (heads_of[None, :] == jnp.arange(HEADS, dtype=jnp.int32)[:, None]
          ).astype(f32)
    s2 = jnp.full((8, NUM_CLASSES), 1.0 / 8.0, f32)

    x_pad = jnp.zeros((NP, F_IN), f32).at[:n].set(x.astype(f32))

    ar = jnp.arange(n, dtype=jnp.int32)
    npad = EP - n - edge_index.shape[1]
    pad_idx = n + jnp.arange(npad, dtype=jnp.int32) % (NP - n)
    src = jnp.concatenate([edge_index[0].astype(jnp.int32), ar, pad_idx])
    dst = jnp.concatenate([edge_index[1].astype(jnp.int32), ar, pad_idx])

    grid = (NP // R_TC,)

    rec1 = pl.pallas_call(
        _prep1_body,
        grid=grid,
        in_specs=[
            pl.BlockSpec((R_TC, F_IN), lambda i: (i, 0)),
            pl.BlockSpec((F_IN, 128), lambda i: (0, 0)),
        ],
        out_specs=pl.BlockSpec((R_TC, 128), lambda i: (i, 0)),
        out_shape=jax.ShapeDtypeStruct((NP, 128), f32),
    )(x_pad, wrec1)

    part1 = _edge_kernel_l1(src, dst, rec1).reshape(2, NP, 128)

    rec2 = pl.pallas_call(
        _merge1_body,
        grid=grid,
        in_specs=[
            pl.BlockSpec((2, R_TC, 128), lambda i: (0, i, 0)),
            pl.BlockSpec((HEADS, C1), lambda i: (0, 0)),
            pl.BlockSpec((1, C1), lambda i: (0, 0)),
            pl.BlockSpec((C1, 128), lambda i: (0, 0)),
        ],
        out_specs=pl.BlockSpec((R_TC, 128), lambda i: (i, 0)),
        out_shape=jax.ShapeDtypeStruct((NP, 128), f32),
    )(part1, s1, b1p, w2cat)

    part2 = _edge_kernel_l2(src, dst, rec2).reshape(2, NP, 128)

    out_full = pl.pallas_call(
        _merge2_body,
        grid=grid,
        in_specs=[
            pl.BlockSpec((2, R_TC, 128), lambda i: (0, i, 0)),
            pl.BlockSpec((8, NUM_CLASSES), lambda i: (0, 0)),
            pl.BlockSpec((1, NUM_CLASSES), lambda i: (0, 0)),
        ],
        out_specs=pl.BlockSpec((R_TC, NUM_CLASSES), lambda i: (i, 0)),
        out_shape=jax.ShapeDtypeStruct((NP, NUM_CLASSES), f32),
    )(part2, s2, b2row)

    return out_full[:n]
